# baseline (speedup 1.0000x reference)
import functools

import jax
import jax.numpy as jnp
from jax import lax
from jax.experimental import pallas as pl
from jax.experimental.pallas import tpu as pltpu
from jax.experimental.pallas import tpu_sc as plsc

B, H, S = 128, 3, 197
HS = H * S
NSLAB = B * H
K = 300
KPAD = 304

NC, NS, L = 2, 16, 16
NW = NC * NS
TRIPS = NSLAB // NW
NGRP = (S + L - 1) // L

_mesh = plsc.VectorSubcoreMesh(core_axis_name="c", subcore_axis_name="s")


@functools.partial(
    pl.kernel,
    mesh=_mesh,
    compiler_params=pltpu.CompilerParams(needs_layout_passes=False),
    out_type=jax.ShapeDtypeStruct((NSLAB, S, S), jnp.float32),
    scratch_types=[
        pltpu.VMEM((608,), jnp.float32),
        pltpu.VMEM((KPAD,), jnp.int32),
        pltpu.VMEM((KPAD,), jnp.int32),
        pltpu.VMEM((S, S), jnp.float32),
        pltpu.VMEM((S, S), jnp.float32),
    ],
)
def _sc_body(x_hbm, ti0_hbm, ti1_hbm, out_hbm, mask_v, ti0_v, ti1_v, in_v, out_v):
    wid = lax.axis_index("s") * NC + lax.axis_index("c")
    lane = lax.iota(jnp.int32, L)

    zero16 = jnp.zeros((L,), jnp.float32)
    for k in range(608 // L):
        mask_v[pl.ds(k * L, L)] = zero16
    pltpu.sync_copy(ti0_hbm, ti0_v)
    pltpu.sync_copy(ti1_hbm, ti1_v)
    one16 = jnp.ones((L,), jnp.float32)
    for k in range(KPAD // L):
        i0 = ti0_v[pl.ds(k * L, L)]
        i1 = ti1_v[pl.ds(k * L, L)]
        idx = i0 * S + i1
        valid = lane < (K - k * L)
        plsc.store_scatter(mask_v, [idx], one16, mask=valid)

    inv_s = jnp.float32(1.0 / S)
    col0 = [L * j for j in range(S // L)] + [S - L]
    tail_new = lane >= (L - S % L)

    def row_body(r, hbase197):
        rb = lax.broadcast(r, (L,))
        m = plsc.load_gather(mask_v, [hbase197 + rb])
        vs = [in_v[r, pl.ds(c, L)] for c in col0]
        es = [jnp.exp(v) for v in vs]
        acc = jnp.where(tail_new, es[-1], jnp.float32(0.0))
        for e in es[:-1]:
            acc = acc + e
        for d in (1, 2, 4, 8):
            acc = acc + jnp.take_along_axis(acc, lane ^ d, axis=0)
        a = m / acc
        cm = (one16 - m) * inv_s
        for c, v, e in zip(col0, vs, es):
            out_v[r, pl.ds(c, L)] = e * a + (v * cm) * v
        return hbase197

    def trip(t, _):
        slab = wid + NW * t
        hbase197 = lax.rem(lax.broadcast(slab, (L,)), H) * S
        pltpu.sync_copy(x_hbm.at[slab], in_v)
        plsc.parallel_loop(0, S, unroll=2, carry=hbase197)(row_body)
        pltpu.sync_copy(out_v, out_hbm.at[slab])
        return 0

    lax.fori_loop(0, TRIPS, trip, 0)


def kernel(attn_weights, true_idx0, true_idx1, false_idx0, false_idx1):
    x = attn_weights.reshape(NSLAB, S, S)
    ti0 = jnp.pad(true_idx0.astype(jnp.int32), (0, KPAD - K))
    ti1 = jnp.pad(true_idx1.astype(jnp.int32), (0, KPAD - K))
    out = _sc_body(x, ti0, ti1)
    return out.reshape(attn_weights.shape)


# speedup vs baseline: 2.2720x; 2.2720x over previous
import functools

import jax
import jax.numpy as jnp
from jax import lax
from jax.experimental import pallas as pl
from jax.experimental.pallas import tpu as pltpu
from jax.experimental.pallas import tpu_sc as plsc

B, H, S = 128, 3, 197
HS = H * S
K = 300
KPAD = 304

NC, NS, L = 2, 16, 16
NW = NC * NS
TRIPS = (HS + NW - 1) // NW
NCH = B // L

_mesh = plsc.VectorSubcoreMesh(core_axis_name="c", subcore_axis_name="s")


@functools.partial(
    pl.kernel,
    mesh=_mesh,
    compiler_params=pltpu.CompilerParams(needs_layout_passes=False),
    out_type=jax.ShapeDtypeStruct((HS, S, B), jnp.float32),
    scratch_types=[
        pltpu.VMEM((608,), jnp.float32),
        pltpu.VMEM((KPAD,), jnp.int32),
        pltpu.VMEM((KPAD,), jnp.int32),
        pltpu.VMEM((S, B), jnp.float32),
        pltpu.VMEM((S, B), jnp.float32),
    ],
)
def _sc_body(x_hbm, ti0_hbm, ti1_hbm, out_hbm, mask_v, ti0_v, ti1_v, in_v, out_v):
    wid = lax.axis_index("s") * NC + lax.axis_index("c")
    lane = lax.iota(jnp.int32, L)

    zero16 = jnp.zeros((L,), jnp.float32)
    for k in range(608 // L):
        mask_v[pl.ds(k * L, L)] = zero16
    pltpu.sync_copy(ti0_hbm, ti0_v)
    pltpu.sync_copy(ti1_hbm, ti1_v)
    one16 = jnp.ones((L,), jnp.float32)
    for k in range(KPAD // L):
        i0 = ti0_v[pl.ds(k * L, L)]
        i1 = ti1_v[pl.ds(k * L, L)]
        idx = i0 * S + i1
        valid = lane < (K - k * L)
        plsc.store_scatter(mask_v, [idx], one16, mask=valid)

    inv_s = jnp.float32(1.0 / S)

    def soft():
        for c in range(NCH):
            c0 = c * L

            def accum(k, acc):
                e = jnp.exp(in_v[k, pl.ds(c0, L)])
                out_v[k, pl.ds(c0, L)] = e
                return acc + e

            acc = plsc.parallel_loop(
                0, S, unroll=8, carry=jnp.zeros((L,), jnp.float32))(accum)
            inv = one16 / acc

            @plsc.parallel_loop(0, S, unroll=8)
            def _(k):
                out_v[k, pl.ds(c0, L)] = out_v[k, pl.ds(c0, L)] * inv

    def sq():
        for c in range(NCH):
            c0 = c * L

            @plsc.parallel_loop(0, S, unroll=8)
            def _(k):
                v = in_v[k, pl.ds(c0, L)]
                out_v[k, pl.ds(c0, L)] = (v * inv_s) * v

    def trip(t, _):
        blk = wid + NW * t

        @pl.when(blk < HS)
        def _():
            pltpu.sync_copy(x_hbm.at[blk], in_v)
            is_soft = plsc.load_gather(mask_v, [lax.broadcast(blk, (L,))])[0] > 0.5
            lax.cond(is_soft, soft, sq)
            pltpu.sync_copy(out_v, out_hbm.at[blk])

        return 0

    lax.fori_loop(0, TRIPS, trip, 0)


def kernel(attn_weights, true_idx0, true_idx1, false_idx0, false_idx1):
    x = jnp.transpose(attn_weights, (1, 2, 3, 0)).reshape(HS, S, B)
    ti0 = jnp.pad(true_idx0.astype(jnp.int32), (0, KPAD - K))
    ti1 = jnp.pad(true_idx1.astype(jnp.int32), (0, KPAD - K))
    out = _sc_body(x, ti0, ti1)
    return jnp.transpose(out.reshape(H, S, S, B), (3, 0, 1, 2))


# speedup vs baseline: 3.1109x; 1.3693x over previous
import functools

import jax
import jax.numpy as jnp
from jax import lax
from jax.experimental import pallas as pl
from jax.experimental.pallas import tpu as pltpu
from jax.experimental.pallas import tpu_sc as plsc

B, H, S = 128, 3, 197
HS = H * S
K = 300
KPAD = 304

NC, NS, L = 2, 16, 16
NW = NC * NS
TRIPS = (HS + NW - 1) // NW
NCH = B // L

_mesh = plsc.VectorSubcoreMesh(core_axis_name="c", subcore_axis_name="s")


@functools.partial(
    pl.kernel,
    mesh=_mesh,
    compiler_params=pltpu.CompilerParams(needs_layout_passes=False),
    out_type=jax.ShapeDtypeStruct((HS, S, B), jnp.float32),
    scratch_types=[
        pltpu.VMEM((608,), jnp.float32),
        pltpu.VMEM((KPAD,), jnp.int32),
        pltpu.VMEM((KPAD,), jnp.int32),
        pltpu.VMEM((S, B), jnp.float32),
        pltpu.VMEM((S, B), jnp.float32),
        pltpu.VMEM((S, B), jnp.float32),
        pltpu.VMEM((S, B), jnp.float32),
        pltpu.SemaphoreType.DMA,
        pltpu.SemaphoreType.DMA,
        pltpu.SemaphoreType.DMA,
        pltpu.SemaphoreType.DMA,
    ],
)
def _sc_body(x_hbm, ti0_hbm, ti1_hbm, out_hbm, mask_v, ti0_v, ti1_v,
             in_v0, in_v1, out_v0, out_v1, sin0, sin1, sout0, sout1):
    wid = lax.axis_index("s") * NC + lax.axis_index("c")
    lane = lax.iota(jnp.int32, L)

    zero16 = jnp.zeros((L,), jnp.float32)
    for k in range(608 // L):
        mask_v[pl.ds(k * L, L)] = zero16
    pltpu.sync_copy(ti0_hbm, ti0_v)
    pltpu.sync_copy(ti1_hbm, ti1_v)
    one16 = jnp.ones((L,), jnp.float32)
    for k in range(KPAD // L):
        i0 = ti0_v[pl.ds(k * L, L)]
        i1 = ti1_v[pl.ds(k * L, L)]
        idx = i0 * S + i1
        valid = lane < (K - k * L)
        plsc.store_scatter(mask_v, [idx], one16, mask=valid)

    inv_s = jnp.float32(1.0 / S)

    def make_soft(in_v, out_v):
        def soft():
            for c in range(NCH):
                c0 = c * L

                def accum(k, acc):
                    e = jnp.exp(in_v[k, pl.ds(c0, L)])
                    out_v[k, pl.ds(c0, L)] = e
                    return acc + e

                acc = plsc.parallel_loop(
                    0, S, unroll=8, carry=jnp.zeros((L,), jnp.float32))(accum)
                inv = one16 / acc

                @plsc.parallel_loop(0, S, unroll=8)
                def _(k):
                    out_v[k, pl.ds(c0, L)] = out_v[k, pl.ds(c0, L)] * inv

        return soft

    def make_sq(in_v, out_v):
        def sq():
            for c in range(NCH):
                c0 = c * L

                @plsc.parallel_loop(0, S, unroll=8)
                def _(k):
                    v = in_v[k, pl.ds(c0, L)]
                    out_v[k, pl.ds(c0, L)] = (v * inv_s) * v

        return sq

    bufs = [
        (in_v0, out_v0, sin0, sout0),
        (in_v1, out_v1, sin1, sout1),
    ]
    fns = [(make_soft(i, o), make_sq(i, o)) for i, o, _, _ in bufs]

    def start_in(blk, p):
        pltpu.make_async_copy(x_hbm.at[blk], bufs[p][0], bufs[p][2]).start()

    start_in(wid, 0)
    start_in(wid + NW, 1)

    def step(blk, p):
        in_v, out_v, sin, sout = bufs[p]
        prev = blk - 2 * NW

        @pl.when(jnp.logical_and(prev >= 0, prev < HS))
        def _():
            pltpu.make_async_copy(x_hbm.at[0], out_v, sout).wait()

        @pl.when(blk < HS)
        def _():
            pltpu.make_async_copy(x_hbm.at[0], in_v, sin).wait()
            is_soft = plsc.load_gather(mask_v, [lax.broadcast(blk, (L,))])[0] > 0.5
            lax.cond(is_soft, fns[p][0], fns[p][1])
            pltpu.make_async_copy(out_v, out_hbm.at[blk], sout).start()

        nxt = blk + 2 * NW

        @pl.when(nxt < HS)
        def _():
            start_in(nxt, p)

    def trip(t, _):
        blk = wid + 2 * NW * t
        step(blk, 0)
        step(blk + NW, 1)
        return 0

    lax.fori_loop(0, (TRIPS + 1) // 2, trip, 0)

    last0 = wid + NW * (TRIPS - 1)

    @pl.when(last0 < HS)
    def _():
        pltpu.make_async_copy(x_hbm.at[0], out_v0, sout0).wait()


def kernel(attn_weights, true_idx0, true_idx1, false_idx0, false_idx1):
    x = jnp.transpose(attn_weights, (1, 2, 3, 0)).reshape(HS, S, B)
    ti0 = jnp.pad(true_idx0.astype(jnp.int32), (0, KPAD - K))
    ti1 = jnp.pad(true_idx1.astype(jnp.int32), (0, KPAD - K))
    out = _sc_body(x, ti0, ti1)
    return jnp.transpose(out.reshape(H, S, S, B), (3, 0, 1, 2))
